# baseline (speedup 1.0000x reference)
import functools

import jax
import jax.numpy as jnp
from jax import lax
from jax.experimental import pallas as pl
from jax.experimental.pallas import tpu as pltpu
from jax.experimental.pallas import tpu_sc as plsc

D = 768
NC = 2
NS = 16
NW = NC * NS
LANES = 16
K = 32


@functools.lru_cache(maxsize=None)
def _make_sc_kernel(total_rows: int, seqlen: int):
    rows_per_w = total_rows // NW
    n_chunks = rows_per_w // K
    w_per_seq = seqlen // rows_per_w
    mesh = plsc.VectorSubcoreMesh(core_axis_name="c", subcore_axis_name="s")

    @functools.partial(
        pl.kernel,
        mesh=mesh,
        out_type=jax.ShapeDtypeStruct((total_rows, D), jnp.float32),
        scratch_types=[
            pltpu.VMEM((rows_per_w,), jnp.int32),
            pltpu.VMEM((K, D), jnp.float32),
            pltpu.VMEM((K, D), jnp.float32),
            pltpu.SemaphoreType.DMA,
        ],
    )
    def k(ids_hbm, table_hbm, pos_hbm, out_hbm, idx_v, rows_v, pos_v, sem):
        wid = lax.axis_index("s") * NC + lax.axis_index("c")
        base = wid * rows_per_w
        pos_base = (wid % w_per_seq) * rows_per_w
        pltpu.sync_copy(ids_hbm.at[pl.ds(base, rows_per_w)], idx_v)

        def chunk_body(c, carry):
            off = pl.multiple_of(c * K, K)
            pltpu.async_copy(
                table_hbm.at[idx_v.at[pl.ds(off, K)]], rows_v, sem
            ).wait()
            pltpu.sync_copy(pos_hbm.at[pl.ds(pos_base + off, K)], pos_v)

            def row_body(r, carry2):
                for j in range(D // LANES):
                    rows_v[r, pl.ds(j * LANES, LANES)] += (
                        pos_v[r, pl.ds(j * LANES, LANES)]
                    )
                return carry2

            lax.fori_loop(0, K, row_body, None)
            pltpu.sync_copy(rows_v, out_hbm.at[pl.ds(base + off, K)])
            return carry

        lax.fori_loop(0, n_chunks, chunk_body, None)

    return k


def kernel(input_ids, word_embeddings, position_embeddings):
    batch, seqlen = input_ids.shape
    total_rows = batch * seqlen
    ids_flat = input_ids.reshape(total_rows).astype(jnp.int32)
    out = _make_sc_kernel(total_rows, seqlen)(
        ids_flat, word_embeddings, position_embeddings
    )
    return out.reshape(batch, seqlen, D)


# speedup vs baseline: 1.9076x; 1.9076x over previous
import functools

import jax
import jax.numpy as jnp
from jax import lax
from jax.experimental import pallas as pl
from jax.experimental.pallas import tpu as pltpu
from jax.experimental.pallas import tpu_sc as plsc

D = 768
NC = 2
NS = 16
NW = NC * NS
LANES = 16
K = 32
NB = 4


@functools.lru_cache(maxsize=None)
def _make_sc_kernel(batch: int, seqlen: int):
    assert batch == NB
    seq_per_w = seqlen // NW
    n_chunks = seq_per_w // K
    mesh = plsc.VectorSubcoreMesh(core_axis_name="c", subcore_axis_name="s")
    total_rows = batch * seqlen

    @functools.partial(
        pl.kernel,
        mesh=mesh,
        out_type=jax.ShapeDtypeStruct((total_rows, D), jnp.float32),
        scratch_types=[
            pltpu.VMEM((batch * seq_per_w,), jnp.int32),
            pltpu.VMEM((K, D), jnp.float32),
            pltpu.VMEM((K, D), jnp.float32),
            pltpu.VMEM((K, D), jnp.float32),
            pltpu.VMEM((K, D), jnp.float32),
            pltpu.VMEM((K, D), jnp.float32),
        ] + [pltpu.SemaphoreType.DMA] * (2 * NB),
    )
    def k(ids_hbm, table_hbm, pos_hbm, out_hbm, idx_v,
          r0, r1, r2, r3, pos_v,
          g0, g1, g2, g3, s0, s1, s2, s3):
        bufs = [r0, r1, r2, r3]
        gsems = [g0, g1, g2, g3]
        ssems = [s0, s1, s2, s3]
        wid = lax.axis_index("s") * NC + lax.axis_index("c")
        seq_base = wid * seq_per_w

        for b in range(batch):
            pltpu.sync_copy(
                ids_hbm.at[pl.ds(b * seqlen + seq_base, seq_per_w)],
                idx_v.at[pl.ds(b * seq_per_w, seq_per_w)],
            )

        def issue_gather(c_dyn, b_u):
            off = pl.multiple_of(b_u * seq_per_w + c_dyn * K, K)
            pltpu.async_copy(
                table_hbm.at[idx_v.at[pl.ds(off, K)]], bufs[b_u], gsems[b_u]
            )

        def wait_gather(b_u):
            pltpu.make_async_copy(
                table_hbm.at[pl.ds(0, K)], bufs[b_u], gsems[b_u]
            ).wait()

        def wait_store(b_u):
            pltpu.make_async_copy(
                bufs[b_u], out_hbm.at[pl.ds(0, K)], ssems[b_u]
            ).wait()

        issue_gather(0, 0)
        issue_gather(0, 1)

        def group(c, carry):
            pltpu.sync_copy(
                pos_hbm.at[pl.ds(seq_base + pl.multiple_of(c * K, K), K)],
                pos_v,
            )
            for b in range(batch):
                wait_gather(b)

                def row_body(r, carry2, _buf=bufs[b]):
                    for j in range(D // LANES):
                        _buf[r, pl.ds(j * LANES, LANES)] += (
                            pos_v[r, pl.ds(j * LANES, LANES)]
                        )
                    return carry2

                lax.fori_loop(0, K, row_body, None)

                row_off = pl.multiple_of(
                    b * seqlen + seq_base + c * K, K
                )
                pltpu.async_copy(
                    bufs[b], out_hbm.at[pl.ds(row_off, K)], ssems[b]
                )

                nb = (b + 2) % NB
                if b < 2:
                    @pl.when(c > 0)
                    def _(nb=nb):
                        wait_store(nb)

                    issue_gather(c, b + 2)
                else:
                    @pl.when(c < n_chunks - 1)
                    def _(c=c, nb=nb):
                        wait_store(nb)
                        issue_gather(c + 1, nb)
            return carry

        lax.fori_loop(0, n_chunks, group, None)

        for b in range(batch):
            wait_store(b)

    return k


def kernel(input_ids, word_embeddings, position_embeddings):
    batch, seqlen = input_ids.shape
    ids_flat = input_ids.reshape(batch * seqlen).astype(jnp.int32)
    out = _make_sc_kernel(batch, seqlen)(
        ids_flat, word_embeddings, position_embeddings
    )
    return out.reshape(batch, seqlen, D)


# speedup vs baseline: 2.0585x; 1.0791x over previous
import functools

import jax
import jax.numpy as jnp
from jax import lax
from jax.experimental import pallas as pl
from jax.experimental.pallas import tpu as pltpu
from jax.experimental.pallas import tpu_sc as plsc

D = 768
NC = 2
NS = 16
NW = NC * NS
LANES = 16
K = 16
NB = 8
PF = 4


@functools.lru_cache(maxsize=None)
def _make_sc_kernel(batch: int, seqlen: int):
    assert batch == 4
    seq_per_w = seqlen // NW
    n_chunks = seq_per_w // K
    n_groups = n_chunks // 2
    n_tasks = n_chunks * batch
    mesh = plsc.VectorSubcoreMesh(core_axis_name="c", subcore_axis_name="s")
    total_rows = batch * seqlen

    @functools.partial(
        pl.kernel,
        mesh=mesh,
        out_type=jax.ShapeDtypeStruct((total_rows, D), jnp.float32),
        scratch_types=(
            [pltpu.VMEM((batch * seq_per_w,), jnp.int32)]
            + [pltpu.VMEM((K, D), jnp.float32)] * NB
            + [pltpu.VMEM((K, D), jnp.float32)] * 2
            + [pltpu.SemaphoreType.DMA] * (2 * NB + 2)
        ),
    )
    def k(ids_hbm, table_hbm, pos_hbm, out_hbm, idx_v, *scr):
        bufs = list(scr[:NB])
        poss = list(scr[NB:NB + 2])
        gsems = list(scr[NB + 2:2 * NB + 2])
        ssems = list(scr[2 * NB + 2:3 * NB + 2])
        psems = list(scr[3 * NB + 2:3 * NB + 4])
        wid = lax.axis_index("s") * NC + lax.axis_index("c")
        seq_base = wid * seq_per_w

        for b in range(batch):
            pltpu.sync_copy(
                ids_hbm.at[pl.ds(b * seqlen + seq_base, seq_per_w)],
                idx_v.at[pl.ds(b * seq_per_w, seq_per_w)],
            )

        def issue_gather(c_dyn, b_stat, buf_i):
            off = pl.multiple_of(b_stat * seq_per_w + c_dyn * K, K)
            pltpu.async_copy(
                table_hbm.at[idx_v.at[pl.ds(off, K)]],
                bufs[buf_i], gsems[buf_i],
            )

        def wait_gather(buf_i):
            pltpu.make_async_copy(
                table_hbm.at[pl.ds(0, K)], bufs[buf_i], gsems[buf_i]
            ).wait()

        def wait_store(buf_i):
            pltpu.make_async_copy(
                bufs[buf_i], out_hbm.at[pl.ds(0, K)], ssems[buf_i]
            ).wait()

        def issue_pos(c_dyn, par):
            pltpu.async_copy(
                pos_hbm.at[pl.ds(seq_base + pl.multiple_of(c_dyn * K, K), K)],
                poss[par], psems[par],
            )

        def wait_pos(par):
            pltpu.make_async_copy(
                pos_hbm.at[pl.ds(0, K)], poss[par], psems[par]
            ).wait()

        issue_pos(0, 0)
        for i in range(PF):
            issue_gather(0, i, i)

        def group(g, carry):
            for i in range(NB):
                par = i // 4
                b = i % 4
                c = 2 * g + par
                if i == 0:
                    wait_pos(0)
                    issue_pos(2 * g + 1, 1)
                if i == 4:
                    wait_pos(1)

                    @pl.when(g < n_groups - 1)
                    def _():
                        issue_pos(2 * g + 2, 0)

                wait_gather(i)

                def row_body(r, carry2, _buf=bufs[i], _pos=poss[par]):
                    for j in range(D // LANES):
                        plsc.addupdate(
                            _buf.at[r, pl.ds(j * LANES, LANES)],
                            _pos[r, pl.ds(j * LANES, LANES)],
                        )
                    return carry2

                lax.fori_loop(0, K, row_body, None)

                row_off = pl.multiple_of(b * seqlen + seq_base + c * K, K)
                pltpu.async_copy(
                    bufs[i], out_hbm.at[pl.ds(row_off, K)], ssems[i]
                )

                nb_i = (i + PF) % NB
                nc_ = 2 * g + 1 + par
                if i < PF:
                    @pl.when(g > 0)
                    def _(nb_i=nb_i):
                        wait_store(nb_i)

                    issue_gather(nc_, b, nb_i)
                else:
                    @pl.when(g < n_groups - 1)
                    def _(nc_=nc_, b=b, nb_i=nb_i):
                        wait_store(nb_i)
                        issue_gather(nc_, b, nb_i)
            return carry

        lax.fori_loop(0, n_groups, group, None)

        for i in range(NB):
            wait_store(i)

    return k


def kernel(input_ids, word_embeddings, position_embeddings):
    batch, seqlen = input_ids.shape
    ids_flat = input_ids.reshape(batch * seqlen).astype(jnp.int32)
    out = _make_sc_kernel(batch, seqlen)(
        ids_flat, word_embeddings, position_embeddings
    )
    return out.reshape(batch, seqlen, D)
